# baseline (speedup 1.0000x reference)
import functools

import jax
import jax.numpy as jnp
from jax import lax
from jax.experimental import pallas as pl
from jax.experimental.pallas import tpu as pltpu
from jax.experimental.pallas import tpu_sc as plsc

NUM_FIELDS = 26
VOCAB = 100000
EMBED_DIM = 32
BATCH = 4096
DENSE_NUM = 13

_NC = 2
_NS = 16
_NW = _NC * _NS
_BF = BATCH * NUM_FIELDS
_RPW = _BF // _NW
_CHUNK = 128
_NCHUNK = _RPW // _CHUNK
_BPW = BATCH // _NW

_mesh = plsc.VectorSubcoreMesh(core_axis_name="c", subcore_axis_name="s")


@functools.partial(
    pl.kernel,
    mesh=_mesh,
    out_type=jax.ShapeDtypeStruct((_BF, EMBED_DIM), jnp.float32),
    scratch_types=[
        pltpu.VMEM((_NCHUNK, _CHUNK), jnp.int32),
        pltpu.VMEM((_RPW, EMBED_DIM), jnp.float32),
        pltpu.SemaphoreType.DMA,
    ],
    compiler_params=pltpu.CompilerParams(use_tc_tiling_on_sc=False),
)
def _sc_gather(idx_hbm, tables_hbm, out_hbm, idx_v, rows_v, sem):
    wid = lax.axis_index("s") * _NC + lax.axis_index("c")
    base_r = wid * _RPW
    pltpu.sync_copy(idx_hbm.at[wid], idx_v)
    iota = lax.iota(jnp.int32, 16)

    def adjust(c, carry):
        for s in range(_CHUNK // 16):
            r = base_r + c * _CHUNK + s * 16 + iota
            f = lax.rem(r, NUM_FIELDS)
            idx_v[c, pl.ds(s * 16, 16)] = idx_v[c, pl.ds(s * 16, 16)] + f * VOCAB
        return carry

    lax.fori_loop(0, _NCHUNK, adjust, 0)

    def fire(c, carry):
        pltpu.make_async_copy(
            tables_hbm.at[idx_v.at[c]], rows_v.at[pl.ds(c * _CHUNK, _CHUNK)], sem
        ).start()
        return carry

    lax.fori_loop(0, _NCHUNK, fire, 0)

    def drain(c, carry):
        pltpu.make_async_copy(
            tables_hbm.at[idx_v.at[c]], rows_v.at[pl.ds(c * _CHUNK, _CHUNK)], sem
        ).wait()
        return carry

    lax.fori_loop(0, _NCHUNK, drain, 0)
    pltpu.sync_copy(rows_v, out_hbm.at[pl.ds(base_r, _RPW)])


def _dense_body(x_ref, w_ref, b_ref, o_ref):
    o_ref[...] = lax.dot_general(
        x_ref[...], w_ref[...], (((1,), (1,)), ((), ())),
        preferred_element_type=jnp.float32,
    ) + b_ref[...]


def _dense_tc(x, w, b2d):
    return pl.pallas_call(
        _dense_body,
        out_shape=jax.ShapeDtypeStruct((BATCH, EMBED_DIM), jnp.float32),
    )(x, w, b2d)


def kernel(sparse_indices, dense_features, tables, W, b):
    idx2d = sparse_indices.reshape(_NW, _NCHUNK, _CHUNK)
    tflat = tables.reshape(NUM_FIELDS * VOCAB, EMBED_DIM)
    rows = _sc_gather(idx2d, tflat)
    dense_emb = _dense_tc(dense_features, W, b[None, :])
    return jnp.concatenate(
        [rows.reshape(BATCH, NUM_FIELDS * EMBED_DIM), dense_emb], axis=-1
    )


# speedup vs baseline: 3.2951x; 3.2951x over previous
import functools

import jax
import jax.numpy as jnp
from jax import lax
from jax.experimental import pallas as pl
from jax.experimental.pallas import tpu as pltpu
from jax.experimental.pallas import tpu_sc as plsc

NUM_FIELDS = 26
VOCAB = 100000
EMBED_DIM = 32
BATCH = 4096
DENSE_NUM = 13

_NC = 2
_NS = 16
_BW = 4096
_NFULL = VOCAB // _BW
_TAILS = _NFULL * _BW
_TAILW = 1664
_LASTS = _TAILS + _TAILW
_NBKT = _NFULL + 2

_mesh = plsc.VectorSubcoreMesh(core_axis_name="c", subcore_axis_name="s")
_iota16 = lambda: lax.iota(jnp.int32, 16)


@functools.partial(
    pl.kernel,
    mesh=_mesh,
    out_type=jax.ShapeDtypeStruct((NUM_FIELDS * EMBED_DIM, BATCH), jnp.float32),
    scratch_types=[
        pltpu.VMEM((EMBED_DIM, BATCH // EMBED_DIM), jnp.int32),
        pltpu.VMEM((4224,), jnp.int32),
        pltpu.VMEM((4224,), jnp.int32),
        pltpu.VMEM((8, _BW), jnp.float32),
        pltpu.VMEM((8, _BW), jnp.float32),
        pltpu.VMEM((8, BATCH), jnp.float32),
        pltpu.VMEM((EMBED_DIM, 128), jnp.float32),
        pltpu.SMEM((_NBKT + 1,), jnp.int32),
        pltpu.SemaphoreType.DMA,
        pltpu.SemaphoreType.DMA,
    ],
    compiler_params=pltpu.CompilerParams(
        use_tc_tiling_on_sc=True, needs_layout_passes=False),
)
def _sc_gather(idx_hbm, tbl_hbm, last_hbm, out_hbm, idx_v, bidx_v, bpos_v,
               ch0, ch1, orow_v, last_v, boff_s, sem0, sem1):
    wid = lax.axis_index("s") * _NC + lax.axis_index("c")
    f = wid

    @pl.when(wid < NUM_FIELDS)
    def _body():
        pltpu.sync_copy(idx_hbm.at[f], idx_v)
        pltpu.sync_copy(last_hbm.at[f], last_v)
        iota = _iota16()

        def _bucket(k, off):
            def _row(r, off):
                for s in range(8):
                    v = idx_v[r, pl.ds(s * 16, 16)]
                    bid = lax.shift_right_logical(v, 12) + jnp.where(
                        v >= _LASTS, 1, 0)
                    m = bid == k
                    cnt = jnp.sum(jnp.where(m, 1, 0))

                    @pl.when(cnt > 0)
                    def _():
                        pos = r * 128 + s * 16 + iota
                        plsc.store_compressed(bidx_v.at[pl.ds(off, 16)], v, mask=m)
                        plsc.store_compressed(bpos_v.at[pl.ds(off, 16)], pos, mask=m)
                    off = off + cnt
                return off
            off = lax.fori_loop(0, EMBED_DIM, _row, off)
            boff_s[k + 1] = off
            return off

        boff_s[0] = 0
        lax.fori_loop(0, _NBKT, _bucket, 0)

        def _chunk_src(d8, k, width):
            return tbl_hbm.at[f, pl.ds(d8 * 8, 8), pl.ds(k * _BW, width)]

        def _process(k, ch, base, dlo):
            start = boff_s[k]
            end = boff_s[k + 1]

            def _vec(v, carry):
                o = start + v * 16
                m = (o + iota) < end
                lidx = bidx_v[pl.ds(o, 16)] - base
                pos = bpos_v[pl.ds(o, 16)]
                for d in range(8):
                    dsp = jnp.full((16,), dlo + d, jnp.int32)
                    osp = jnp.full((16,), d, jnp.int32)
                    val = plsc.load_gather(ch, [dsp, lidx], mask=m)
                    plsc.store_scatter(orow_v, [osp, pos], val, mask=m)
                return carry

            nvec = lax.div(end - start + 15, 16)
            lax.fori_loop(0, nvec, _vec, 0)

        for d8 in range(4):
            pltpu.make_async_copy(_chunk_src(d8, 0, _BW), ch0, sem0).start()
            pltpu.make_async_copy(_chunk_src(d8, 1, _BW), ch1, sem1).start()

            def _pair(kk, carry):
                k0 = kk * 2
                pltpu.make_async_copy(_chunk_src(d8, k0, _BW), ch0, sem0).wait()
                _process(k0, ch0, k0 * _BW, 0)

                @pl.when(k0 + 2 < _NFULL)
                def _():
                    pltpu.make_async_copy(
                        _chunk_src(d8, k0 + 2, _BW), ch0, sem0).start()
                k1 = k0 + 1
                pltpu.make_async_copy(_chunk_src(d8, k1, _BW), ch1, sem1).wait()
                _process(k1, ch1, k1 * _BW, 0)

                @pl.when(k1 + 2 < _NFULL)
                def _():
                    pltpu.make_async_copy(
                        _chunk_src(d8, k1 + 2, _BW), ch1, sem1).start()
                return carry

            lax.fori_loop(0, _NFULL // 2, _pair, 0)
            pltpu.sync_copy(_chunk_src(d8, _NFULL, _TAILW),
                            ch0.at[:, pl.ds(0, _TAILW)])
            _process(_NFULL, ch0, _TAILS, 0)
            _process(_NFULL + 1, last_v, VOCAB - 128, d8 * 8)
            pltpu.sync_copy(orow_v, out_hbm.at[pl.ds(f * EMBED_DIM + d8 * 8, 8)])


def _dense_body(x_ref, w_ref, b_ref, o_ref):
    o_ref[...] = lax.dot_general(
        x_ref[...], w_ref[...], (((1,), (1,)), ((), ())),
        preferred_element_type=jnp.float32,
    ) + b_ref[...]


def _dense_tc(x, w, b2d):
    return pl.pallas_call(
        _dense_body,
        out_shape=jax.ShapeDtypeStruct((BATCH, EMBED_DIM), jnp.float32),
    )(x, w, b2d)


def kernel(sparse_indices, dense_features, tables, W, b):
    tbl_t = jnp.transpose(tables, (0, 2, 1))
    last_t = jnp.transpose(tables[:, VOCAB - 128:, :], (0, 2, 1))
    idx3d = jnp.transpose(sparse_indices, (1, 0)).reshape(
        NUM_FIELDS, EMBED_DIM, BATCH // EMBED_DIM)
    sparse_t = _sc_gather(idx3d, tbl_t, last_t)
    dense_emb = _dense_tc(dense_features, W, b[None, :])
    return jnp.concatenate(
        [jnp.transpose(sparse_t, (1, 0)), dense_emb], axis=-1
    )


# speedup vs baseline: 3.8821x; 1.1781x over previous
import functools

import jax
import jax.numpy as jnp
from jax import lax
from jax.experimental import pallas as pl
from jax.experimental.pallas import tpu as pltpu
from jax.experimental.pallas import tpu_sc as plsc

NUM_FIELDS = 26
VOCAB = 100000
EMBED_DIM = 32
BATCH = 4096
DENSE_NUM = 13

_NC = 2
_NS = 16
_BW = 4096
_NFULL = VOCAB // _BW
_TAILS = _NFULL * _BW
_TAILW = 1664
_LASTS = _TAILS + _TAILW
_NBKT = _NFULL + 2

_mesh = plsc.VectorSubcoreMesh(core_axis_name="c", subcore_axis_name="s")
_iota16 = lambda: lax.iota(jnp.int32, 16)


@functools.partial(
    pl.kernel,
    mesh=_mesh,
    out_type=jax.ShapeDtypeStruct((NUM_FIELDS * EMBED_DIM, BATCH), jnp.float32),
    scratch_types=[
        pltpu.VMEM((EMBED_DIM, BATCH // EMBED_DIM), jnp.int32),
        pltpu.VMEM((4224,), jnp.int32),
        pltpu.VMEM((4224,), jnp.int32),
        pltpu.VMEM((8, _BW), jnp.float32),
        pltpu.VMEM((8, _BW), jnp.float32),
        pltpu.VMEM((8, BATCH), jnp.float32),
        pltpu.VMEM((EMBED_DIM, 128), jnp.float32),
        pltpu.SMEM((_NBKT + 1,), jnp.int32),
        pltpu.SemaphoreType.DMA,
        pltpu.SemaphoreType.DMA,
    ],
    compiler_params=pltpu.CompilerParams(
        use_tc_tiling_on_sc=True, needs_layout_passes=False),
)
def _sc_gather(idx_hbm, tbl_hbm, last_hbm, out_hbm, idx_v, bidx_v, bpos_v,
               ch0, ch1, orow_v, last_v, boff_s, sem0, sem1):
    wid = lax.axis_index("s") * _NC + lax.axis_index("c")
    f = wid

    @pl.when(wid < NUM_FIELDS)
    def _body():
        pltpu.sync_copy(idx_hbm.at[f], idx_v)
        pltpu.sync_copy(last_hbm.at[f], last_v)
        iota = _iota16()

        def _bucket(k, off_vec):
            def _row(r, off_vec):
                for s in range(8):
                    v = idx_v[r, pl.ds(s * 16, 16)]
                    bid = lax.shift_right_logical(v, 12) + jnp.where(
                        v >= _LASTS, 1, 0)
                    m = bid == k
                    cnt = plsc.all_reduce_population_count(m)
                    rank = plsc.cumsum(jnp.where(m, 1, 0)) - 1
                    dst = off_vec + rank
                    pos = r * 128 + s * 16 + iota
                    plsc.store_scatter(bidx_v, [dst], v, mask=m)
                    plsc.store_scatter(bpos_v, [dst], pos, mask=m)
                    off_vec = off_vec + cnt
                return off_vec
            off_vec = lax.fori_loop(0, EMBED_DIM, _row, off_vec)
            boff_s[k + 1] = jnp.max(off_vec)
            return off_vec

        boff_s[0] = 0
        lax.fori_loop(0, _NBKT, _bucket, jnp.zeros((16,), jnp.int32))

        def _chunk_src(d8, k, width):
            return tbl_hbm.at[f, pl.ds(d8 * 8, 8), pl.ds(k * _BW, width)]

        def _process(k, ch, base, dlo):
            start = boff_s[k]
            end = boff_s[k + 1]

            def _vec(v, carry):
                o = start + v * 16
                m = (o + iota) < end
                lidx = bidx_v[pl.ds(o, 16)] - base
                pos = bpos_v[pl.ds(o, 16)]
                for d in range(8):
                    dsp = jnp.full((16,), dlo + d, jnp.int32)
                    osp = jnp.full((16,), d, jnp.int32)
                    val = plsc.load_gather(ch, [dsp, lidx], mask=m)
                    plsc.store_scatter(orow_v, [osp, pos], val, mask=m)
                return carry

            nvec = lax.div(end - start + 15, 16)
            lax.fori_loop(0, nvec, _vec, 0)

        for d8 in range(4):
            pltpu.make_async_copy(_chunk_src(d8, 0, _BW), ch0, sem0).start()
            pltpu.make_async_copy(_chunk_src(d8, 1, _BW), ch1, sem1).start()

            def _pair(kk, carry):
                k0 = kk * 2
                pltpu.make_async_copy(_chunk_src(d8, k0, _BW), ch0, sem0).wait()
                _process(k0, ch0, k0 * _BW, 0)

                @pl.when(k0 + 2 < _NFULL)
                def _():
                    pltpu.make_async_copy(
                        _chunk_src(d8, k0 + 2, _BW), ch0, sem0).start()
                k1 = k0 + 1
                pltpu.make_async_copy(_chunk_src(d8, k1, _BW), ch1, sem1).wait()
                _process(k1, ch1, k1 * _BW, 0)

                @pl.when(k1 + 2 < _NFULL)
                def _():
                    pltpu.make_async_copy(
                        _chunk_src(d8, k1 + 2, _BW), ch1, sem1).start()
                return carry

            lax.fori_loop(0, _NFULL // 2, _pair, 0)
            pltpu.sync_copy(_chunk_src(d8, _NFULL, _TAILW),
                            ch0.at[:, pl.ds(0, _TAILW)])
            _process(_NFULL, ch0, _TAILS, 0)
            _process(_NFULL + 1, last_v, VOCAB - 128, d8 * 8)
            pltpu.sync_copy(orow_v, out_hbm.at[pl.ds(f * EMBED_DIM + d8 * 8, 8)])


def _dense_body(x_ref, w_ref, b_ref, o_ref):
    o_ref[...] = lax.dot_general(
        x_ref[...], w_ref[...], (((1,), (1,)), ((), ())),
        preferred_element_type=jnp.float32,
    ) + b_ref[...]


def _dense_tc(x, w, b2d):
    return pl.pallas_call(
        _dense_body,
        out_shape=jax.ShapeDtypeStruct((BATCH, EMBED_DIM), jnp.float32),
    )(x, w, b2d)


def kernel(sparse_indices, dense_features, tables, W, b):
    tbl_t = jnp.transpose(tables, (0, 2, 1))
    last_t = jnp.transpose(tables[:, VOCAB - 128:, :], (0, 2, 1))
    idx3d = jnp.transpose(sparse_indices, (1, 0)).reshape(
        NUM_FIELDS, EMBED_DIM, BATCH // EMBED_DIM)
    sparse_t = _sc_gather(idx3d, tbl_t, last_t)
    dense_emb = _dense_tc(dense_features, W, b[None, :])
    return jnp.concatenate(
        [jnp.transpose(sparse_t, (1, 0)), dense_emb], axis=-1
    )
